# Initial kernel scaffold; baseline (speedup 1.0000x reference)
import jax
import jax.numpy as jnp
from jax.experimental import pallas as pl


def kernel(shape_xyz, skel_xyz, skel_nori):
    raise NotImplementedError("write your pallas kernel here")


# speedup vs baseline: 10.2904x; 10.2904x over previous
import functools

import jax
import jax.numpy as jnp
from jax import lax
from jax.experimental import pallas as pl
from jax.experimental.pallas import tpu as pltpu
from jax.experimental.pallas import tpu_sc as plsc

B = 8
N = 4096
M = 256
L = 16
NC = 2
NS = 16
Q = 4
NQ = N // Q
NK = NQ // L
BIG = 3.0e38
EPS = 1e-12


def _rsqrt(x):
    i = plsc.bitcast(x, jnp.int32)
    i = 0x5F3759DF - lax.shift_right_logical(i, 1)
    y = plsc.bitcast(i, jnp.float32)
    xh = x * 0.5
    for _ in range(3):
        y = y * (1.5 - xh * y * y)
    return y


def _sqrt(x):
    return x * _rsqrt(x)


_MESH = plsc.VectorSubcoreMesh(core_axis_name="c", subcore_axis_name="s",
                               num_cores=NC, num_subcores=NS)


@functools.partial(
    pl.kernel,
    out_type=jax.ShapeDtypeStruct((B * L,), jnp.float32),
    mesh=_MESH,
    compiler_params=pltpu.CompilerParams(needs_layout_passes=False),
    scratch_types=dict(
        sd=pltpu.VMEM_SHARED((NS * 2 * M,), jnp.float32),
        si=pltpu.VMEM_SHARED((NS * 2 * M,), jnp.int32),
        sc1=pltpu.VMEM_SHARED((NS * L,), jnp.float32),
        pts_v=pltpu.VMEM((3 * NQ,), jnp.float32),
        skel_v=pltpu.VMEM((3 * M,), jnp.float32),
        colmin_v=pltpu.VMEM((NQ,), jnp.float32),
        ni_v=pltpu.VMEM((NQ,), jnp.int32),
        d_v=pltpu.VMEM((2 * M,), jnp.float32),
        i_v=pltpu.VMEM((2 * M,), jnp.int32),
        c_v=pltpu.VMEM((L,), jnp.float32),
        rd_v=pltpu.VMEM((Q * 2 * M,), jnp.float32),
        ri_v=pltpu.VMEM((Q * 2 * M,), jnp.int32),
        c4_v=pltpu.VMEM((Q * L,), jnp.float32),
        nrm_v=pltpu.VMEM((3 * N,), jnp.float32),
        nori_v=pltpu.VMEM((3 * M,), jnp.float32),
        out_v=pltpu.VMEM((L,), jnp.float32),
    ),
)
def _sc_loss(pts_hbm, nrm_hbm, skel_hbm, nori_hbm, out_hbm,
             sd, si, sc1, pts_v, skel_v, colmin_v, ni_v, d_v, i_v, c_v,
             rd_v, ri_v, c4_v, nrm_v, nori_v, out_v):
    c = lax.axis_index("c")
    s = lax.axis_index("s")
    b = c * (B // NC) + s // Q
    q = s % Q
    n0 = q * NQ

    for r in range(3):
        pltpu.sync_copy(pts_hbm.at[pl.ds((b * 3 + r) * N + n0, NQ)],
                        pts_v.at[pl.ds(r * NQ, NQ)])
        pltpu.sync_copy(skel_hbm.at[pl.ds((b * 3 + r) * M, M)],
                        skel_v.at[pl.ds(r * M, M)])

    lane = lax.iota(jnp.int32, L)
    m0 = lane == 0
    big = jnp.full((L,), BIG, jnp.float32)
    zi = jnp.zeros((L,), jnp.int32)

    def init_k(k, carry):
        colmin_v[pl.ds(k * L, L)] = big
        ni_v[pl.ds(k * L, L)] = lane + (n0 + k * L)
        return carry

    lax.fori_loop(0, NK, init_k, 0)

    def m_body(m, carry):
        idxm = jnp.full((L,), m, jnp.int32)
        qx = plsc.load_gather(skel_v, [idxm])
        qy = plsc.load_gather(skel_v, [idxm + M])
        qz = plsc.load_gather(skel_v, [idxm + 2 * M])

        def n_body(k, st):
            d1, d2, i1, i2 = st
            o = k * L
            dx = pts_v[pl.ds(o, L)] - qx
            dy = pts_v[pl.ds(NQ + o, L)] - qy
            dz = pts_v[pl.ds(2 * NQ + o, L)] - qz
            v = dx * dx + dy * dy + dz * dz
            colmin_v[pl.ds(o, L)] = jnp.minimum(colmin_v[pl.ds(o, L)], v)
            ni = ni_v[pl.ds(o, L)]
            c1 = v < d1
            c2 = v < d2
            d2n = jnp.where(c1, d1, jnp.where(c2, v, d2))
            i2n = jnp.where(c1, i1, jnp.where(c2, ni, i2))
            return (jnp.where(c1, v, d1), d2n, jnp.where(c1, ni, i1), i2n)

        d1, d2, i1, i2 = lax.fori_loop(0, NK, n_body, (big, big, zi, zi),
                                       unroll=2)
        sd1, sj1, sd2, sj2 = lax.sort([d1, i1, d2, i2], dimension=0,
                                      num_keys=1)
        d_first = sd1[0]
        i_first = sj1[0]
        d_a = sd1[1]
        i_a = sj1[1]
        d_b = sd2[0]
        i_b = sj2[0]
        use_b = d_b < d_a
        d_sec = jnp.where(use_b, d_b, d_a)
        i_sec = jnp.where(use_b, i_b, i_a)
        plsc.store_scatter(d_v, [idxm],
                           jnp.full((L,), d_first, jnp.float32), mask=m0)
        plsc.store_scatter(d_v, [idxm + M],
                           jnp.full((L,), d_sec, jnp.float32), mask=m0)
        plsc.store_scatter(i_v, [idxm],
                           jnp.full((L,), i_first, jnp.int32), mask=m0)
        plsc.store_scatter(i_v, [idxm + M],
                           jnp.full((L,), i_sec, jnp.int32), mask=m0)
        return carry

    lax.fori_loop(0, M, m_body, 0)

    def cd1_k(k, acc):
        return acc + _sqrt(colmin_v[pl.ds(k * L, L)] + EPS)

    c_v[...] = lax.fori_loop(0, NK, cd1_k, jnp.zeros((L,), jnp.float32))

    pltpu.sync_copy(c_v, sc1.at[pl.ds(s * L, L)])
    pltpu.sync_copy(d_v, sd.at[pl.ds(s * 2 * M, 2 * M)])
    pltpu.sync_copy(i_v, si.at[pl.ds(s * 2 * M, 2 * M)])

    plsc.subcore_barrier()

    @pl.when(q == 0)
    def _merge():
        for r in range(3):
            pltpu.sync_copy(nrm_hbm.at[pl.ds((b * 3 + r) * N, N)],
                            nrm_v.at[pl.ds(r * N, N)])
            pltpu.sync_copy(nori_hbm.at[pl.ds((b * 3 + r) * M, M)],
                            nori_v.at[pl.ds(r * M, M)])
        for qq in range(Q):
            pltpu.sync_copy(sd.at[pl.ds((s + qq) * 2 * M, 2 * M)],
                            rd_v.at[pl.ds(qq * 2 * M, 2 * M)])
            pltpu.sync_copy(si.at[pl.ds((s + qq) * 2 * M, 2 * M)],
                            ri_v.at[pl.ds(qq * 2 * M, 2 * M)])
            pltpu.sync_copy(sc1.at[pl.ds((s + qq) * L, L)],
                            c4_v.at[pl.ds(qq * L, L)])

        acc_cd2 = jnp.zeros((L,), jnp.float32)
        acc_nl = jnp.zeros((L,), jnp.float32)
        for mc in range(M // L):
            o = mc * L
            d1 = rd_v[pl.ds(o, L)]
            d2 = rd_v[pl.ds(M + o, L)]
            i1 = ri_v[pl.ds(o, L)]
            i2 = ri_v[pl.ds(M + o, L)]
            for qq in range(1, Q):
                base = qq * 2 * M
                e1 = rd_v[pl.ds(base + o, L)]
                e2 = rd_v[pl.ds(base + M + o, L)]
                j1 = ri_v[pl.ds(base + o, L)]
                j2 = ri_v[pl.ds(base + M + o, L)]
                cf = e1 < d1
                ca = e2 < d1
                cb = e1 < d2
                sa_d = jnp.where(ca, e2, d1)
                sa_i = jnp.where(ca, j2, i1)
                sb_d = jnp.where(cb, e1, d2)
                sb_i = jnp.where(cb, j1, i2)
                d2 = jnp.where(cf, sa_d, sb_d)
                i2 = jnp.where(cf, sa_i, sb_i)
                d1 = jnp.where(cf, e1, d1)
                i1 = jnp.where(cf, j1, i1)
            acc_cd2 = acc_cd2 + _sqrt(d1 + EPS)
            nx1 = plsc.load_gather(nrm_v, [i1])
            ny1 = plsc.load_gather(nrm_v, [i1 + N])
            nz1 = plsc.load_gather(nrm_v, [i1 + 2 * N])
            nx2 = plsc.load_gather(nrm_v, [i2])
            ny2 = plsc.load_gather(nrm_v, [i2 + N])
            nz2 = plsc.load_gather(nrm_v, [i2 + 2 * N])
            ox = nori_v[pl.ds(o, L)]
            oy = nori_v[pl.ds(M + o, L)]
            oz = nori_v[pl.ds(2 * M + o, L)]
            dot1 = jnp.abs(nx1 * ox + ny1 * oy + nz1 * oz)
            dot2 = jnp.abs(nx2 * ox + ny2 * oy + nz2 * oz)
            acc_nl = acc_nl + 0.5 * (dot1 + dot2)

        cd1_b = jnp.sum(c4_v[pl.ds(0, L)] + c4_v[pl.ds(L, L)]
                        + c4_v[pl.ds(2 * L, L)] + c4_v[pl.ds(3 * L, L)])
        cd2_b = jnp.sum(acc_cd2)
        nl_b = jnp.sum(acc_nl)
        total_b = 1e-4 * (cd1_b + cd2_b) + (1e-3 / B) * nl_b
        out_v[...] = jnp.full((L,), total_b, jnp.float32)
        pltpu.sync_copy(out_v, out_hbm.at[pl.ds(b * L, L)])


def kernel(shape_xyz, skel_xyz, skel_nori):
    pts = jnp.transpose(shape_xyz[:, :, :3], (0, 2, 1)).reshape(-1)
    nrm = jnp.transpose(shape_xyz[:, :, 3:6], (0, 2, 1)).reshape(-1)
    skel = jnp.transpose(skel_xyz, (0, 2, 1)).reshape(-1)
    nori = jnp.transpose(skel_nori, (0, 2, 1)).reshape(-1)
    out = _sc_loss(pts, nrm, skel, nori)
    return jnp.sum(out[::L])


# speedup vs baseline: 17.2223x; 1.6736x over previous
import functools

import jax
import jax.numpy as jnp
from jax import lax
from jax.experimental import pallas as pl
from jax.experimental.pallas import tpu as pltpu
from jax.experimental.pallas import tpu_sc as plsc

B = 8
N = 4096
M = 256
L = 16
NC = 2
NS = 16
Q = 4
NQ = N // Q
NK = NQ // L
IMASK = 0xFFF
IBIG = 0x7FFFFFFF
BIG = 3.0e38
EPS = 1e-12


def _rsqrt(x):
    i = plsc.bitcast(x, jnp.int32)
    i = 0x5F3759DF - lax.shift_right_logical(i, 1)
    y = plsc.bitcast(i, jnp.float32)
    xh = x * 0.5
    for _ in range(3):
        y = y * (1.5 - xh * y * y)
    return y


def _sqrt(x):
    return x * _rsqrt(x)


def _merge2(a1, a2, b1, b2):
    return (jnp.minimum(a1, b1),
            jnp.minimum(jnp.maximum(a1, b1), jnp.minimum(a2, b2)))


_MESH = plsc.VectorSubcoreMesh(core_axis_name="c", subcore_axis_name="s",
                               num_cores=NC, num_subcores=NS)


@functools.partial(
    pl.kernel,
    out_type=jax.ShapeDtypeStruct((B * L,), jnp.float32),
    mesh=_MESH,
    compiler_params=pltpu.CompilerParams(needs_layout_passes=False),
    scratch_types=dict(
        sk=pltpu.VMEM_SHARED((NS * 2 * M,), jnp.int32),
        sc1=pltpu.VMEM_SHARED((NS * L,), jnp.float32),
        pts_v=pltpu.VMEM((3 * NQ,), jnp.float32),
        skel_v=pltpu.VMEM((3 * M,), jnp.float32),
        colmin_v=pltpu.VMEM((NQ,), jnp.float32),
        ni_v=pltpu.VMEM((NQ,), jnp.int32),
        k_v=pltpu.VMEM((2 * M,), jnp.int32),
        c_v=pltpu.VMEM((L,), jnp.float32),
        rk_v=pltpu.VMEM((Q * 2 * M,), jnp.int32),
        c4_v=pltpu.VMEM((Q * L,), jnp.float32),
        nrm_v=pltpu.VMEM((3 * N,), jnp.float32),
        nori_v=pltpu.VMEM((3 * M,), jnp.float32),
        out_v=pltpu.VMEM((L,), jnp.float32),
    ),
)
def _sc_loss(pts_hbm, nrm_hbm, skel_hbm, nori_hbm, out_hbm,
             sk, sc1, pts_v, skel_v, colmin_v, ni_v, k_v, c_v,
             rk_v, c4_v, nrm_v, nori_v, out_v):
    c = lax.axis_index("c")
    s = lax.axis_index("s")
    b = c * (B // NC) + s // Q
    q = s % Q
    n0 = q * NQ

    for r in range(3):
        pltpu.sync_copy(pts_hbm.at[pl.ds((b * 3 + r) * N + n0, NQ)],
                        pts_v.at[pl.ds(r * NQ, NQ)])
        pltpu.sync_copy(skel_hbm.at[pl.ds((b * 3 + r) * M, M)],
                        skel_v.at[pl.ds(r * M, M)])

    lane = lax.iota(jnp.int32, L)
    m0 = lane == 0
    big = jnp.full((L,), BIG, jnp.float32)
    ibig = jnp.full((L,), IBIG, jnp.int32)

    def init_k(k, carry):
        colmin_v[pl.ds(k * L, L)] = big
        ni_v[pl.ds(k * L, L)] = lane + (n0 + k * L)
        return carry

    lax.fori_loop(0, NK, init_k, 0)

    def m_body(mi, carry):
        m = mi * 2
        idxm = jnp.full((L,), m, jnp.int32)
        idxm2 = idxm + 1
        qxa = plsc.load_gather(skel_v, [idxm])
        qya = plsc.load_gather(skel_v, [idxm + M])
        qza = plsc.load_gather(skel_v, [idxm + 2 * M])
        qxb = plsc.load_gather(skel_v, [idxm2])
        qyb = plsc.load_gather(skel_v, [idxm2 + M])
        qzb = plsc.load_gather(skel_v, [idxm2 + 2 * M])

        def n_body(k, st):
            k1a, k2a, k1b, k2b = st
            o = k * L
            px = pts_v[pl.ds(o, L)]
            py = pts_v[pl.ds(NQ + o, L)]
            pz = pts_v[pl.ds(2 * NQ + o, L)]
            ni = ni_v[pl.ds(o, L)]
            dxa = px - qxa
            dya = py - qya
            dza = pz - qza
            va = dxa * dxa + dya * dya + dza * dza
            dxb = px - qxb
            dyb = py - qyb
            dzb = pz - qzb
            vb = dxb * dxb + dyb * dyb + dzb * dzb
            colmin_v[pl.ds(o, L)] = jnp.minimum(colmin_v[pl.ds(o, L)],
                                                jnp.minimum(va, vb))
            kva = plsc.bitcast(va, jnp.int32) | ni
            kvb = plsc.bitcast(vb, jnp.int32) | ni
            c1a = kva < k1a
            c2a = kva < k2a
            k2a = jnp.where(c1a, k1a, jnp.where(c2a, kva, k2a))
            k1a = jnp.where(c1a, kva, k1a)
            c1b = kvb < k1b
            c2b = kvb < k2b
            k2b = jnp.where(c1b, k1b, jnp.where(c2b, kvb, k2b))
            k1b = jnp.where(c1b, kvb, k1b)
            return (k1a, k2a, k1b, k2b)

        k1a, k2a, k1b, k2b = lax.fori_loop(0, NK, n_body,
                                           (ibig, ibig, ibig, ibig),
                                           unroll=2)
        for (kk1, kk2, im) in ((k1a, k2a, idxm), (k1b, k2b, idxm2)):
            srt = lax.sort([kk1], dimension=0, num_keys=1)[0]
            first = srt[0]
            cand_a = srt[1]
            cand_b = jnp.min(jnp.where(kk1 == first, kk2, IBIG))
            second = jnp.minimum(cand_a, cand_b)
            plsc.store_scatter(k_v, [im],
                               jnp.full((L,), first, jnp.int32), mask=m0)
            plsc.store_scatter(k_v, [im + M],
                               jnp.full((L,), second, jnp.int32), mask=m0)
        return carry

    lax.fori_loop(0, M // 2, m_body, 0)

    def cd1_k(k, acc):
        return acc + _sqrt(colmin_v[pl.ds(k * L, L)] + EPS)

    c_v[...] = lax.fori_loop(0, NK, cd1_k, jnp.zeros((L,), jnp.float32))

    pltpu.sync_copy(c_v, sc1.at[pl.ds(s * L, L)])
    pltpu.sync_copy(k_v, sk.at[pl.ds(s * 2 * M, 2 * M)])

    plsc.subcore_barrier()

    @pl.when(q == 0)
    def _merge():
        for r in range(3):
            pltpu.sync_copy(nrm_hbm.at[pl.ds((b * 3 + r) * N, N)],
                            nrm_v.at[pl.ds(r * N, N)])
            pltpu.sync_copy(nori_hbm.at[pl.ds((b * 3 + r) * M, M)],
                            nori_v.at[pl.ds(r * M, M)])
        for qq in range(Q):
            pltpu.sync_copy(sk.at[pl.ds((s + qq) * 2 * M, 2 * M)],
                            rk_v.at[pl.ds(qq * 2 * M, 2 * M)])
            pltpu.sync_copy(sc1.at[pl.ds((s + qq) * L, L)],
                            c4_v.at[pl.ds(qq * L, L)])

        acc_cd2 = jnp.zeros((L,), jnp.float32)
        acc_nl = jnp.zeros((L,), jnp.float32)
        for mc in range(M // L):
            o = mc * L
            k1 = rk_v[pl.ds(o, L)]
            k2 = rk_v[pl.ds(M + o, L)]
            for qq in range(1, Q):
                base = qq * 2 * M
                e1 = rk_v[pl.ds(base + o, L)]
                e2 = rk_v[pl.ds(base + M + o, L)]
                k1, k2 = _merge2(k1, k2, e1, e2)
            i1 = k1 & IMASK
            i2 = k2 & IMASK
            d1 = plsc.bitcast(k1 & ~IMASK, jnp.float32)
            acc_cd2 = acc_cd2 + _sqrt(d1 + EPS)
            nx1 = plsc.load_gather(nrm_v, [i1])
            ny1 = plsc.load_gather(nrm_v, [i1 + N])
            nz1 = plsc.load_gather(nrm_v, [i1 + 2 * N])
            nx2 = plsc.load_gather(nrm_v, [i2])
            ny2 = plsc.load_gather(nrm_v, [i2 + N])
            nz2 = plsc.load_gather(nrm_v, [i2 + 2 * N])
            ox = nori_v[pl.ds(o, L)]
            oy = nori_v[pl.ds(M + o, L)]
            oz = nori_v[pl.ds(2 * M + o, L)]
            dot1 = jnp.abs(nx1 * ox + ny1 * oy + nz1 * oz)
            dot2 = jnp.abs(nx2 * ox + ny2 * oy + nz2 * oz)
            acc_nl = acc_nl + 0.5 * (dot1 + dot2)

        cd1_b = jnp.sum(c4_v[pl.ds(0, L)] + c4_v[pl.ds(L, L)]
                        + c4_v[pl.ds(2 * L, L)] + c4_v[pl.ds(3 * L, L)])
        cd2_b = jnp.sum(acc_cd2)
        nl_b = jnp.sum(acc_nl)
        total_b = 1e-4 * (cd1_b + cd2_b) + (1e-3 / B) * nl_b
        out_v[...] = jnp.full((L,), total_b, jnp.float32)
        pltpu.sync_copy(out_v, out_hbm.at[pl.ds(b * L, L)])


def kernel(shape_xyz, skel_xyz, skel_nori):
    pts = jnp.transpose(shape_xyz[:, :, :3], (0, 2, 1)).reshape(-1)
    nrm = jnp.transpose(shape_xyz[:, :, 3:6], (0, 2, 1)).reshape(-1)
    skel = jnp.transpose(skel_xyz, (0, 2, 1)).reshape(-1)
    nori = jnp.transpose(skel_nori, (0, 2, 1)).reshape(-1)
    out = _sc_loss(pts, nrm, skel, nori)
    return jnp.sum(out[::L])


# speedup vs baseline: 22.3276x; 1.2964x over previous
import functools

import jax
import jax.numpy as jnp
from jax import lax
from jax.experimental import pallas as pl
from jax.experimental.pallas import tpu as pltpu
from jax.experimental.pallas import tpu_sc as plsc

B = 8
N = 4096
M = 256
L = 16
NC = 2
NS = 16
Q = 4
NQ = N // Q
NK = NQ // L
IMASK = 0xFFF
IBIG = 0x7FFFFFFF
BIG = 3.0e38
EPS = 1e-12


def _rsqrt(x):
    i = plsc.bitcast(x, jnp.int32)
    i = 0x5F3759DF - lax.shift_right_logical(i, 1)
    y = plsc.bitcast(i, jnp.float32)
    xh = x * 0.5
    for _ in range(3):
        y = y * (1.5 - xh * y * y)
    return y


def _sqrt(x):
    return x * _rsqrt(x)


def _merge2(a1, a2, b1, b2):
    return (jnp.minimum(a1, b1),
            jnp.minimum(jnp.maximum(a1, b1), jnp.minimum(a2, b2)))


_MESH = plsc.VectorSubcoreMesh(core_axis_name="c", subcore_axis_name="s",
                               num_cores=NC, num_subcores=NS)


@functools.partial(
    pl.kernel,
    out_type=jax.ShapeDtypeStruct((B * L,), jnp.float32),
    mesh=_MESH,
    compiler_params=pltpu.CompilerParams(needs_layout_passes=False),
    scratch_types=dict(
        sk=pltpu.VMEM_SHARED((NS * 2 * M,), jnp.int32),
        sc1=pltpu.VMEM_SHARED((NS * L,), jnp.float32),
        pts_v=pltpu.VMEM((3 * NQ,), jnp.float32),
        skel_v=pltpu.VMEM((3 * M,), jnp.float32),
        colmin_v=pltpu.VMEM((NQ,), jnp.float32),
        k_v=pltpu.VMEM((2 * M,), jnp.int32),
        c_v=pltpu.VMEM((L,), jnp.float32),
        rk_v=pltpu.VMEM((Q * 2 * M,), jnp.int32),
        c4_v=pltpu.VMEM((Q * L,), jnp.float32),
        nrm_v=pltpu.VMEM((3 * N,), jnp.float32),
        nori_v=pltpu.VMEM((3 * M,), jnp.float32),
        out_v=pltpu.VMEM((L,), jnp.float32),
    ),
)
def _sc_loss(pts_hbm, nrm_hbm, skel_hbm, nori_hbm, out_hbm,
             sk, sc1, pts_v, skel_v, colmin_v, k_v, c_v,
             rk_v, c4_v, nrm_v, nori_v, out_v):
    c = lax.axis_index("c")
    s = lax.axis_index("s")
    b = c * (B // NC) + s // Q
    q = s % Q
    n0 = q * NQ

    for r in range(3):
        pltpu.sync_copy(pts_hbm.at[pl.ds((b * 3 + r) * N + n0, NQ)],
                        pts_v.at[pl.ds(r * NQ, NQ)])
        pltpu.sync_copy(skel_hbm.at[pl.ds((b * 3 + r) * M, M)],
                        skel_v.at[pl.ds(r * M, M)])

    lane = lax.iota(jnp.int32, L)
    m0 = lane == 0
    big = jnp.full((L,), BIG, jnp.float32)
    ibig = jnp.full((L,), IBIG, jnp.int32)
    lane0 = lane + n0

    def init_k(k, carry):
        colmin_v[pl.ds(k * L, L)] = big
        return carry

    lax.fori_loop(0, NK, init_k, 0)

    def m_body(mi, carry):
        m = mi * 2
        idxm = jnp.full((L,), m, jnp.int32)
        idxm2 = idxm + 1
        qxa = plsc.load_gather(skel_v, [idxm])
        qya = plsc.load_gather(skel_v, [idxm + M])
        qza = plsc.load_gather(skel_v, [idxm + 2 * M])
        qxb = plsc.load_gather(skel_v, [idxm2])
        qyb = plsc.load_gather(skel_v, [idxm2 + M])
        qzb = plsc.load_gather(skel_v, [idxm2 + 2 * M])

        k1a = ibig
        k2a = ibig
        k1b = ibig
        k2b = ibig
        for kk in range(NK):
            o = kk * L
            px = pts_v[pl.ds(o, L)]
            py = pts_v[pl.ds(NQ + o, L)]
            pz = pts_v[pl.ds(2 * NQ + o, L)]
            ni = lane0 + o
            dxa = px - qxa
            dya = py - qya
            dza = pz - qza
            va = dxa * dxa + dya * dya + dza * dza
            dxb = px - qxb
            dyb = py - qyb
            dzb = pz - qzb
            vb = dxb * dxb + dyb * dyb + dzb * dzb
            colmin_v[pl.ds(o, L)] = jnp.minimum(colmin_v[pl.ds(o, L)],
                                                jnp.minimum(va, vb))
            kva = plsc.bitcast(va, jnp.int32) | ni
            kvb = plsc.bitcast(vb, jnp.int32) | ni
            k2a = jnp.minimum(k2a, jnp.maximum(k1a, kva))
            k1a = jnp.minimum(k1a, kva)
            k2b = jnp.minimum(k2b, jnp.maximum(k1b, kvb))
            k1b = jnp.minimum(k1b, kvb)

        for (kk1, kk2, im) in ((k1a, k2a, idxm), (k1b, k2b, idxm2)):
            srt = lax.sort([kk1], dimension=0, num_keys=1)[0]
            first = srt[0]
            cand_a = srt[1]
            cand_b = jnp.min(jnp.where(kk1 == first, kk2, IBIG))
            second = jnp.minimum(cand_a, cand_b)
            plsc.store_scatter(k_v, [im],
                               jnp.full((L,), first, jnp.int32), mask=m0)
            plsc.store_scatter(k_v, [im + M],
                               jnp.full((L,), second, jnp.int32), mask=m0)
        return carry

    lax.fori_loop(0, M // 2, m_body, 0)

    def cd1_k(k, acc):
        return acc + _sqrt(colmin_v[pl.ds(k * L, L)] + EPS)

    c_v[...] = lax.fori_loop(0, NK, cd1_k, jnp.zeros((L,), jnp.float32))

    pltpu.sync_copy(c_v, sc1.at[pl.ds(s * L, L)])
    pltpu.sync_copy(k_v, sk.at[pl.ds(s * 2 * M, 2 * M)])

    plsc.subcore_barrier()

    @pl.when(q == 0)
    def _merge():
        for r in range(3):
            pltpu.sync_copy(nrm_hbm.at[pl.ds((b * 3 + r) * N, N)],
                            nrm_v.at[pl.ds(r * N, N)])
            pltpu.sync_copy(nori_hbm.at[pl.ds((b * 3 + r) * M, M)],
                            nori_v.at[pl.ds(r * M, M)])
        for qq in range(Q):
            pltpu.sync_copy(sk.at[pl.ds((s + qq) * 2 * M, 2 * M)],
                            rk_v.at[pl.ds(qq * 2 * M, 2 * M)])
            pltpu.sync_copy(sc1.at[pl.ds((s + qq) * L, L)],
                            c4_v.at[pl.ds(qq * L, L)])

        acc_cd2 = jnp.zeros((L,), jnp.float32)
        acc_nl = jnp.zeros((L,), jnp.float32)
        for mc in range(M // L):
            o = mc * L
            k1 = rk_v[pl.ds(o, L)]
            k2 = rk_v[pl.ds(M + o, L)]
            for qq in range(1, Q):
                base = qq * 2 * M
                e1 = rk_v[pl.ds(base + o, L)]
                e2 = rk_v[pl.ds(base + M + o, L)]
                k1, k2 = _merge2(k1, k2, e1, e2)
            i1 = k1 & IMASK
            i2 = k2 & IMASK
            d1 = plsc.bitcast(k1 & ~IMASK, jnp.float32)
            acc_cd2 = acc_cd2 + _sqrt(d1 + EPS)
            nx1 = plsc.load_gather(nrm_v, [i1])
            ny1 = plsc.load_gather(nrm_v, [i1 + N])
            nz1 = plsc.load_gather(nrm_v, [i1 + 2 * N])
            nx2 = plsc.load_gather(nrm_v, [i2])
            ny2 = plsc.load_gather(nrm_v, [i2 + N])
            nz2 = plsc.load_gather(nrm_v, [i2 + 2 * N])
            ox = nori_v[pl.ds(o, L)]
            oy = nori_v[pl.ds(M + o, L)]
            oz = nori_v[pl.ds(2 * M + o, L)]
            dot1 = jnp.abs(nx1 * ox + ny1 * oy + nz1 * oz)
            dot2 = jnp.abs(nx2 * ox + ny2 * oy + nz2 * oz)
            acc_nl = acc_nl + 0.5 * (dot1 + dot2)

        cd1_b = jnp.sum(c4_v[pl.ds(0, L)] + c4_v[pl.ds(L, L)]
                        + c4_v[pl.ds(2 * L, L)] + c4_v[pl.ds(3 * L, L)])
        cd2_b = jnp.sum(acc_cd2)
        nl_b = jnp.sum(acc_nl)
        total_b = 1e-4 * (cd1_b + cd2_b) + (1e-3 / B) * nl_b
        out_v[...] = jnp.full((L,), total_b, jnp.float32)
        pltpu.sync_copy(out_v, out_hbm.at[pl.ds(b * L, L)])


def kernel(shape_xyz, skel_xyz, skel_nori):
    pts = jnp.transpose(shape_xyz[:, :, :3], (0, 2, 1)).reshape(-1)
    nrm = jnp.transpose(shape_xyz[:, :, 3:6], (0, 2, 1)).reshape(-1)
    skel = jnp.transpose(skel_xyz, (0, 2, 1)).reshape(-1)
    nori = jnp.transpose(skel_nori, (0, 2, 1)).reshape(-1)
    out = _sc_loss(pts, nrm, skel, nori)
    return jnp.sum(out[::L])
